# Initial kernel scaffold; baseline (speedup 1.0000x reference)
import jax
import jax.numpy as jnp
from jax.experimental import pallas as pl


def kernel(x, router, shared_gate, shared_up, shared_down, gate, up, down):
    raise NotImplementedError("write your pallas kernel here")


# speedup vs baseline: 1.0238x; 1.0238x over previous
import functools

import jax
import jax.numpy as jnp
from jax import lax
from jax.experimental import pallas as pl
from jax.experimental.pallas import tpu as pltpu
from jax.experimental.pallas import tpu_sc as plsc

N = 4096
D = 1024
E = 8
TOPK = 2
FF = 2048
NP = N * TOPK
TILE = 256
NBLK = (NP + E * TILE) // TILE
PADN = NBLK * TILE

SC_NC = 2
SC_NS = 16
SC_NW = SC_NC * SC_NS
SC_CHUNK = 64


def _silu(v):
    return v * jax.nn.sigmoid(v)


def _shared_router_body(x_ref, xb_ref, sg_ref, su_ref, sd_ref, r_ref,
                        shared_ref, w1_ref, w2_ref, i1_ref, i2_ref):
    xb = xb_ref[...]
    h = _silu(jnp.dot(xb, sg_ref[...], preferred_element_type=jnp.float32))
    h = h * jnp.dot(xb, su_ref[...], preferred_element_type=jnp.float32)
    shared_ref[...] = jnp.dot(h.astype(jnp.bfloat16), sd_ref[...],
                              preferred_element_type=jnp.float32)
    logits = jnp.dot(x_ref[...], r_ref[...], preferred_element_type=jnp.float32)
    m = jnp.max(logits, axis=1, keepdims=True)
    ex = jnp.exp(logits - m)
    probs = ex / jnp.sum(ex, axis=1, keepdims=True)
    idx = lax.broadcasted_iota(jnp.int32, probs.shape, 1)
    w1 = jnp.max(probs, axis=1)
    i1 = jnp.argmax(probs, axis=1).astype(jnp.int32)
    masked = jnp.where(idx == i1[:, None], -1.0, probs)
    w2 = jnp.max(masked, axis=1)
    i2 = jnp.argmax(masked, axis=1).astype(jnp.int32)
    w1_ref[...] = w1[:, None]
    w2_ref[...] = w2[:, None]
    i1_ref[...] = i1[:, None]
    i2_ref[...] = i2[:, None]


def _shared_and_route(xf, xb, router, sg, su, sd):
    TB = 256
    grid = (N // TB,)
    return pl.pallas_call(
        _shared_router_body,
        grid=grid,
        in_specs=[
            pl.BlockSpec((TB, D), lambda i: (i, 0)),
            pl.BlockSpec((TB, D), lambda i: (i, 0)),
            pl.BlockSpec((D, FF), lambda i: (0, 0)),
            pl.BlockSpec((D, FF), lambda i: (0, 0)),
            pl.BlockSpec((FF, D), lambda i: (0, 0)),
            pl.BlockSpec((D, E), lambda i: (0, 0)),
        ],
        out_specs=[
            pl.BlockSpec((TB, D), lambda i: (i, 0)),
            pl.BlockSpec((TB, 1), lambda i: (i, 0)),
            pl.BlockSpec((TB, 1), lambda i: (i, 0)),
            pl.BlockSpec((TB, 1), lambda i: (i, 0)),
            pl.BlockSpec((TB, 1), lambda i: (i, 0)),
        ],
        out_shape=[
            jax.ShapeDtypeStruct((N, D), jnp.float32),
            jax.ShapeDtypeStruct((N, 1), jnp.float32),
            jax.ShapeDtypeStruct((N, 1), jnp.float32),
            jax.ShapeDtypeStruct((N, 1), jnp.int32),
            jax.ShapeDtypeStruct((N, 1), jnp.int32),
        ],
        compiler_params=pltpu.CompilerParams(
            dimension_semantics=("arbitrary",)),
    )(xf, xb, sg, su, sd, router)


def _sc_gather(table, idx, rows_total, d):
    per_w = rows_total // SC_NW
    n_chunks = per_w // SC_CHUNK
    mesh = plsc.VectorSubcoreMesh(core_axis_name="c", subcore_axis_name="s")

    @functools.partial(
        pl.kernel,
        out_type=jax.ShapeDtypeStruct((rows_total, d), table.dtype),
        mesh=mesh,
        scratch_types=[
            pltpu.VMEM((SC_CHUNK,), jnp.int32),
            pltpu.VMEM((SC_CHUNK, d), table.dtype),
            pltpu.SemaphoreType.DMA,
        ],
    )
    def k(table_hbm, idx_hbm, out_hbm, idx_v, rows_v, sem):
        wid = lax.axis_index("c") * SC_NS + lax.axis_index("s")
        base = wid * per_w
        for c in range(n_chunks):
            off = base + c * SC_CHUNK
            pltpu.sync_copy(idx_hbm.at[pl.ds(off, SC_CHUNK)], idx_v)
            pltpu.async_copy(table_hbm.at[idx_v], rows_v, sem).wait()
            pltpu.sync_copy(rows_v, out_hbm.at[pl.ds(off, SC_CHUNK)])

    return k(table, idx)


def _ffn_body(eid_ref, xs_ref, w_ref, g_ref, u_ref, d_ref, ys_ref):
    xs = xs_ref[...].astype(jnp.bfloat16)
    h = _silu(jnp.dot(xs, g_ref[0], preferred_element_type=jnp.float32))
    h = h * jnp.dot(xs, u_ref[0], preferred_element_type=jnp.float32)
    h = h * w_ref[...]
    ys_ref[...] = jnp.dot(h.astype(jnp.bfloat16), d_ref[0],
                          preferred_element_type=jnp.float32)


def _expert_ffn(xs, w_s, blk_eid, gate, up, down):
    grid_spec = pltpu.PrefetchScalarGridSpec(
        num_scalar_prefetch=1,
        grid=(NBLK,),
        in_specs=[
            pl.BlockSpec((TILE, D), lambda i, e: (i, 0)),
            pl.BlockSpec((TILE, 1), lambda i, e: (i, 0)),
            pl.BlockSpec((1, D, FF), lambda i, e: (e[i], 0, 0)),
            pl.BlockSpec((1, D, FF), lambda i, e: (e[i], 0, 0)),
            pl.BlockSpec((1, FF, D), lambda i, e: (e[i], 0, 0)),
        ],
        out_specs=pl.BlockSpec((TILE, D), lambda i, e: (i, 0)),
    )
    return pl.pallas_call(
        _ffn_body,
        grid_spec=grid_spec,
        out_shape=jax.ShapeDtypeStruct((PADN, D), jnp.float32),
        compiler_params=pltpu.CompilerParams(
            dimension_semantics=("arbitrary",)),
    )(blk_eid, xs, w_s, gate, up, down)


def _combine_body(sh_ref, y0_ref, y1_ref, out_ref):
    out_ref[...] = sh_ref[...] + y0_ref[...] + y1_ref[...]


def _combine(shared, yg):
    TB = 256
    half = N // TB
    return pl.pallas_call(
        _combine_body,
        grid=(half,),
        in_specs=[
            pl.BlockSpec((TB, D), lambda i: (i, 0)),
            pl.BlockSpec((TB, D), lambda i: (i, 0)),
            pl.BlockSpec((TB, D), lambda i: (i + half, 0)),
        ],
        out_specs=pl.BlockSpec((TB, D), lambda i: (i, 0)),
        out_shape=jax.ShapeDtypeStruct((N, D), jnp.float32),
        compiler_params=pltpu.CompilerParams(
            dimension_semantics=("arbitrary",)),
    )(shared, yg, yg)


@jax.jit
def kernel(x, router, shared_gate, shared_up, shared_down, gate, up, down):
    B, T, _ = x.shape
    xf = x.reshape(N, D)
    xb = xf.astype(jnp.bfloat16)

    shared, w1, w2, i1, i2 = _shared_and_route(
        xf, xb, router,
        shared_gate.astype(jnp.bfloat16),
        shared_up.astype(jnp.bfloat16),
        shared_down.astype(jnp.bfloat16))

    ei = jnp.concatenate([i1[:, 0], i2[:, 0]])
    wi = jnp.concatenate([w1[:, 0], w2[:, 0]])
    tok = jnp.tile(jnp.arange(N, dtype=jnp.int32), 2)
    order = jnp.argsort(ei)
    e_sorted = ei[order]
    counts = jnp.sum(ei[:, None] == jnp.arange(E)[None, :], axis=0)
    pcounts = ((counts + TILE - 1) // TILE) * TILE
    start = jnp.concatenate([jnp.zeros(1, counts.dtype), jnp.cumsum(counts)[:-1]])
    pstart = jnp.concatenate([jnp.zeros(1, counts.dtype), jnp.cumsum(pcounts)[:-1]])
    rank = jnp.arange(NP, dtype=jnp.int32) - start[e_sorted]
    dest = (pstart[e_sorted] + rank).astype(jnp.int32)
    tok_s = jnp.zeros(PADN, jnp.int32).at[dest].set(tok[order])
    w_s = jnp.zeros((PADN, 1), jnp.float32).at[dest, 0].set(wi[order])
    pairpos = jnp.zeros(NP, jnp.int32).at[order].set(dest)
    bstart = pstart // TILE
    blk = jnp.arange(NBLK, dtype=bstart.dtype)
    blk_eid = (jnp.sum(blk[:, None] >= bstart[None, :], axis=1) - 1).astype(jnp.int32)

    xs = _sc_gather(xf, tok_s, PADN, D)

    ys = _expert_ffn(xs, w_s, blk_eid,
                     gate.astype(jnp.bfloat16),
                     up.astype(jnp.bfloat16),
                     down.astype(jnp.bfloat16))

    yg = _sc_gather(ys, pairpos, NP, D)

    out = _combine(shared, yg)
    return out.reshape(B, T, D)


# speedup vs baseline: 1.0972x; 1.0717x over previous
import functools

import jax
import jax.numpy as jnp
from jax import lax
from jax.experimental import pallas as pl
from jax.experimental.pallas import tpu as pltpu
from jax.experimental.pallas import tpu_sc as plsc

N = 4096
D = 1024
E = 8
TOPK = 2
FF = 2048
NP = N * TOPK
TILE = 256
NBLK = (NP + E * TILE) // TILE
PADN = NBLK * TILE

SC_NC = 2
SC_NS = 16
SC_NW = SC_NC * SC_NS
SC_CHUNK = 64


def _silu(v):
    return v * jax.nn.sigmoid(v)


def _router_body(x_ref, r_ref, w1_ref, w2_ref, i1_ref, i2_ref):
    logits = jnp.dot(x_ref[...], r_ref[...], preferred_element_type=jnp.float32)
    m = jnp.max(logits, axis=1, keepdims=True)
    ex = jnp.exp(logits - m)
    probs = ex / jnp.sum(ex, axis=1, keepdims=True)
    idx = lax.broadcasted_iota(jnp.int32, probs.shape, 1)
    w1 = jnp.max(probs, axis=1)
    i1 = jnp.argmax(probs, axis=1).astype(jnp.int32)
    masked = jnp.where(idx == i1[:, None], -1.0, probs)
    w2 = jnp.max(masked, axis=1)
    i2 = jnp.argmax(masked, axis=1).astype(jnp.int32)
    w1_ref[...] = w1[:, None]
    w2_ref[...] = w2[:, None]
    i1_ref[...] = i1[:, None]
    i2_ref[...] = i2[:, None]


def _route(xf, router):
    TB = 1024
    return pl.pallas_call(
        _router_body,
        grid=(N // TB,),
        in_specs=[
            pl.BlockSpec((TB, D), lambda i: (i, 0)),
            pl.BlockSpec((D, E), lambda i: (0, 0)),
        ],
        out_specs=[pl.BlockSpec((TB, 1), lambda i: (i, 0))] * 4,
        out_shape=[
            jax.ShapeDtypeStruct((N, 1), jnp.float32),
            jax.ShapeDtypeStruct((N, 1), jnp.float32),
            jax.ShapeDtypeStruct((N, 1), jnp.int32),
            jax.ShapeDtypeStruct((N, 1), jnp.int32),
        ],
        compiler_params=pltpu.CompilerParams(
            dimension_semantics=("arbitrary",)),
    )(xf, router)


def _shared_body(xb_ref, sg_ref, su_ref, sd_ref, shared_ref):
    xb = xb_ref[...]
    h = _silu(jnp.dot(xb, sg_ref[...], preferred_element_type=jnp.float32))
    h = h * jnp.dot(xb, su_ref[...], preferred_element_type=jnp.float32)
    shared_ref[...] = jnp.dot(h.astype(jnp.bfloat16), sd_ref[...],
                              preferred_element_type=jnp.float32)


def _shared_ffn(xb, sg, su, sd):
    TB = 256
    return pl.pallas_call(
        _shared_body,
        grid=(N // TB,),
        in_specs=[
            pl.BlockSpec((TB, D), lambda i: (i, 0)),
            pl.BlockSpec((D, FF), lambda i: (0, 0)),
            pl.BlockSpec((D, FF), lambda i: (0, 0)),
            pl.BlockSpec((FF, D), lambda i: (0, 0)),
        ],
        out_specs=pl.BlockSpec((TB, D), lambda i: (i, 0)),
        out_shape=jax.ShapeDtypeStruct((N, D), jnp.float32),
        compiler_params=pltpu.CompilerParams(
            dimension_semantics=("arbitrary",)),
    )(xb, sg, su, sd)


def _sc_gather(table, idx, rows_total, d):
    per_w = rows_total // SC_NW
    n_chunks = per_w // SC_CHUNK
    mesh = plsc.VectorSubcoreMesh(core_axis_name="c", subcore_axis_name="s")

    @functools.partial(
        pl.kernel,
        out_type=jax.ShapeDtypeStruct((rows_total, d), table.dtype),
        mesh=mesh,
        scratch_types=[
            pltpu.VMEM((SC_CHUNK,), jnp.int32),
            pltpu.VMEM((SC_CHUNK, d), table.dtype),
            pltpu.SemaphoreType.DMA,
        ],
    )
    def k(table_hbm, idx_hbm, out_hbm, idx_v, rows_v, sem):
        wid = lax.axis_index("c") * SC_NS + lax.axis_index("s")
        base = wid * per_w
        for c in range(n_chunks):
            off = base + c * SC_CHUNK
            pltpu.sync_copy(idx_hbm.at[pl.ds(off, SC_CHUNK)], idx_v)
            pltpu.async_copy(table_hbm.at[idx_v], rows_v, sem).wait()
            pltpu.sync_copy(rows_v, out_hbm.at[pl.ds(off, SC_CHUNK)])

    return k(table, idx)


def _combine_body(sh_ref, y0_ref, y1_ref, out_ref):
    out_ref[...] = sh_ref[...] + y0_ref[...] + y1_ref[...]


def _combine(shared, yg):
    TB = 256
    half = N // TB
    return pl.pallas_call(
        _combine_body,
        grid=(half,),
        in_specs=[
            pl.BlockSpec((TB, D), lambda i: (i, 0)),
            pl.BlockSpec((TB, D), lambda i: (i, 0)),
            pl.BlockSpec((TB, D), lambda i: (i + half, 0)),
        ],
        out_specs=pl.BlockSpec((TB, D), lambda i: (i, 0)),
        out_shape=jax.ShapeDtypeStruct((N, D), jnp.float32),
        compiler_params=pltpu.CompilerParams(
            dimension_semantics=("arbitrary",)),
    )(shared, yg, yg)


def _ffn_body(eid_ref, xs_ref, w_ref, g_ref, u_ref, d_ref, ys_ref):
    xs = xs_ref[...].astype(jnp.bfloat16)
    h = _silu(jnp.dot(xs, g_ref[0], preferred_element_type=jnp.float32))
    h = h * jnp.dot(xs, u_ref[0], preferred_element_type=jnp.float32)
    h = h * w_ref[...]
    ys_ref[...] = jnp.dot(h.astype(jnp.bfloat16), d_ref[0],
                          preferred_element_type=jnp.float32)


def _expert_ffn(xs, w_s, blk_eid, gate, up, down):
    grid_spec = pltpu.PrefetchScalarGridSpec(
        num_scalar_prefetch=1,
        grid=(NBLK,),
        in_specs=[
            pl.BlockSpec((TILE, D), lambda i, e: (i, 0)),
            pl.BlockSpec((TILE, 1), lambda i, e: (i, 0)),
            pl.BlockSpec((1, D, FF), lambda i, e: (e[i], 0, 0)),
            pl.BlockSpec((1, D, FF), lambda i, e: (e[i], 0, 0)),
            pl.BlockSpec((1, FF, D), lambda i, e: (e[i], 0, 0)),
        ],
        out_specs=pl.BlockSpec((TILE, D), lambda i, e: (i, 0)),
    )
    return pl.pallas_call(
        _ffn_body,
        grid_spec=grid_spec,
        out_shape=jax.ShapeDtypeStruct((PADN, D), jnp.float32),
        compiler_params=pltpu.CompilerParams(
            dimension_semantics=("arbitrary",)),
    )(blk_eid, xs, w_s, gate, up, down)


@jax.jit
def kernel(x, router, shared_gate, shared_up, shared_down, gate, up, down):
    B, T, _ = x.shape
    xf = x.reshape(N, D)
    xb = xf.astype(jnp.bfloat16)

    w1, w2, i1, i2 = _route(xf, router)

    ei = jnp.concatenate([i1[:, 0], i2[:, 0]])
    wi = jnp.concatenate([w1[:, 0], w2[:, 0]])
    onehot = (ei[:, None] == jnp.arange(E, dtype=jnp.int32)[None, :])
    csum = jnp.cumsum(onehot.astype(jnp.int32), axis=0)
    counts = csum[-1]
    rank = jnp.take_along_axis(csum, ei[:, None], axis=1)[:, 0] - 1
    pcounts = ((counts + TILE - 1) // TILE) * TILE
    pstart = jnp.concatenate(
        [jnp.zeros(1, counts.dtype), jnp.cumsum(pcounts)[:-1]])
    dest = (pstart[ei] + rank).astype(jnp.int32)
    tok = jnp.tile(jnp.arange(N, dtype=jnp.int32), 2)
    tok_s = jnp.zeros(PADN, jnp.int32).at[dest].set(tok)
    w_s = jnp.zeros((PADN, 1), jnp.float32).at[dest, 0].set(wi)
    bstart = pstart // TILE
    blk = jnp.arange(NBLK, dtype=bstart.dtype)
    blk_eid = (jnp.sum(blk[:, None] >= bstart[None, :], axis=1) - 1
               ).astype(jnp.int32)

    xs = _sc_gather(xf, tok_s, PADN, D)

    shared = _shared_ffn(xb,
                         shared_gate.astype(jnp.bfloat16),
                         shared_up.astype(jnp.bfloat16),
                         shared_down.astype(jnp.bfloat16))

    ys = _expert_ffn(xs, w_s, blk_eid,
                     gate.astype(jnp.bfloat16),
                     up.astype(jnp.bfloat16),
                     down.astype(jnp.bfloat16))

    yg = _sc_gather(ys, dest, NP, D)

    out = _combine(shared, yg)
    return out.reshape(B, T, D)
